# speedup vs baseline: 12.3108x; 12.3108x over previous
import functools

import jax
import jax.numpy as jnp
from jax.experimental import pallas as pl
from jax.experimental.pallas import tpu as pltpu

C_OUT = 5
K = 3
H_IN, W_IN = 20, 20
H_CONV, W_CONV = H_IN - K + 1, W_IN - K + 1
H_POOL, W_POOL = H_CONV - 1, W_CONV - 1
W_PAD = 24
FEAT_HALF_PAD = C_OUT * H_POOL * W_PAD
FEAT_PAD = 2 * FEAT_HALF_PAD
FEAT = 2 * C_OUT * H_POOL * W_POOL
HID1, HID2, N_CLS = 64, 16, 4


def _round_up(x, m):
    return (x + m - 1) // m * m


def _fused_kernel(x_ref, cw_ref, cb_ref, w1_ref, b1_ref, w2_ref, b2_ref,
                  w3_ref, b3_ref, o_ref):
    s = pl.program_id(0)
    x = x_ref[...]
    tr = x.shape[-1]

    accs = [None] * C_OUT
    for di in range(K):
        for dj in range(K):
            slab = x[:, di:di + H_CONV, dj:dj + W_CONV, :]
            for c in range(C_OUT):
                term = slab * cw_ref[s, c * K * K + di * K + dj]
                accs[c] = term if accs[c] is None else accs[c] + term

    zpad = jnp.zeros((2, H_POOL, W_PAD - W_POOL, tr), jnp.float32)
    pooled = []
    for c in range(C_OUT):
        a = jnp.maximum(accs[c] + cb_ref[s, c], 0.0)
        m1 = jnp.maximum(a[:, :, :W_POOL, :], a[:, :, 1:, :])
        m = jnp.maximum(m1[:, :H_POOL, :, :], m1[:, 1:, :, :])
        pooled.append(jnp.concatenate([m, zpad], axis=2))

    feats = jnp.stack(pooled, axis=1)
    feats = feats.reshape(FEAT_PAD, tr).astype(jnp.bfloat16)

    h = jnp.dot(w1_ref[...], feats, preferred_element_type=jnp.float32)
    h = jnp.maximum(h + b1_ref[...], 0.0)
    h = jnp.dot(w2_ref[...], h.astype(jnp.bfloat16),
                preferred_element_type=jnp.float32)
    h = jnp.maximum(h + b2_ref[...], 0.0)
    o = jnp.dot(w3_ref[...], h.astype(jnp.bfloat16),
                preferred_element_type=jnp.float32)
    o_ref[...] = o + b3_ref[...]


@functools.partial(jax.jit, static_argnames=())
def kernel(x1, x2, conv_w, conv_b, fc1_w, fc1_b, fc2_w, fc2_b, fc3_w, fc3_b):
    n = x1.shape[0]
    rows = n // 2
    TR = 256
    rows_pad = _round_up(rows, TR)

    xs = jnp.stack([x1.reshape(n, H_IN, W_IN), x2.reshape(n, H_IN, W_IN)], 0)
    xs = xs.reshape(2, rows, 2, H_IN, W_IN)
    xs = jnp.transpose(xs, (0, 2, 3, 4, 1))
    xs = jnp.pad(xs, ((0, 0),) * 4 + ((0, rows_pad - rows),))

    w1p = fc1_w.reshape(HID1, 2, C_OUT, H_POOL, W_POOL)
    w1p = jnp.pad(w1p, ((0, 0),) * 4 + ((0, W_PAD - W_POOL),))
    w1p = w1p.reshape(HID1, FEAT_PAD).astype(jnp.bfloat16)
    w2 = fc2_w.astype(jnp.bfloat16)
    w3 = fc3_w.astype(jnp.bfloat16)

    grid = (2, rows_pad // TR)
    flops = 2 * n * (9 * C_OUT * H_CONV * W_CONV
                     + FEAT * HID1 // 2 + HID1 * HID2 // 2 + HID2 * N_CLS // 2)
    bytes_accessed = 4 * (2 * 2 * H_IN * W_IN * rows_pad + N_CLS * 2 * rows_pad)
    oT = pl.pallas_call(
        _fused_kernel,
        out_shape=jax.ShapeDtypeStruct((2, N_CLS, rows_pad), jnp.float32),
        grid=grid,
        in_specs=[
            pl.BlockSpec((None, 2, H_IN, W_IN, TR), lambda s, t: (s, 0, 0, 0, t)),
            pl.BlockSpec(memory_space=pltpu.MemorySpace.SMEM),
            pl.BlockSpec(memory_space=pltpu.MemorySpace.SMEM),
            pl.BlockSpec((HID1, FEAT_PAD), lambda s, t: (0, 0)),
            pl.BlockSpec((HID1, 1), lambda s, t: (0, 0)),
            pl.BlockSpec((HID2, HID1), lambda s, t: (0, 0)),
            pl.BlockSpec((HID2, 1), lambda s, t: (0, 0)),
            pl.BlockSpec((N_CLS, HID2), lambda s, t: (0, 0)),
            pl.BlockSpec((N_CLS, 1), lambda s, t: (0, 0)),
        ],
        out_specs=pl.BlockSpec((None, N_CLS, TR), lambda s, t: (s, 0, t)),
        compiler_params=pltpu.CompilerParams(
            dimension_semantics=("parallel", "parallel")),
        cost_estimate=pl.CostEstimate(flops=flops, transcendentals=0,
                                      bytes_accessed=bytes_accessed),
    )(xs, conv_w, conv_b, w1p, fc1_b, w2, fc2_b, w3, fc3_b)

    o = jnp.transpose(oT[:, :, :rows], (1, 0, 2)).reshape(N_CLS, 2 * rows)
    return o.T


# speedup vs baseline: 13.2546x; 1.0767x over previous
import functools

import jax
import jax.numpy as jnp
import numpy as np
from jax.experimental import pallas as pl
from jax.experimental.pallas import tpu as pltpu

C_OUT = 5
KW = 3
H_IN, W_IN = 20, 20
PIX = H_IN * W_IN
H_CONV, W_CONV = H_IN - KW + 1, W_IN - KW + 1
H_POOL, W_POOL = H_CONV - 1, W_CONV - 1
FRAME = H_CONV * W_CONV
FRAME_PAD = 384
FEAT_HALF = C_OUT * H_POOL * W_POOL
HID1, HID2, N_CLS = 64, 16, 4
BS = 512


def _sel_mats():
    sel = np.zeros((KW, H_IN, H_CONV), np.float32)
    for d in range(KW):
        for o in range(H_CONV):
            sel[d, o + d, o] = 1.0
    return jnp.asarray(sel)


def _branch_kernel(x_ref, wb_ref, cb_ref, w1a_ref, w1b_ref, b1_ref,
                   w2_ref, b2_ref, w3_ref, b3_ref, o_ref):
    x = x_ref[...].astype(jnp.bfloat16)
    ha = None
    hb = None
    for c in range(C_OUT):
        lo = c * FRAME_PAD
        z = jnp.dot(x, wb_ref[:, lo:lo + FRAME_PAD],
                    preferred_element_type=jnp.float32)
        z = jnp.maximum(z + cb_ref[:, lo:lo + FRAME_PAD], 0.0)
        m = jnp.maximum(z, pltpu.roll(z, FRAME_PAD - 1, axis=1))
        m = jnp.maximum(m, pltpu.roll(m, FRAME_PAD - W_CONV, axis=1))
        feats = m.astype(jnp.bfloat16)
        pa = jnp.dot(feats, w1a_ref[lo:lo + FRAME_PAD, :],
                     preferred_element_type=jnp.float32)
        pb = jnp.dot(feats, w1b_ref[lo:lo + FRAME_PAD, :],
                     preferred_element_type=jnp.float32)
        ha = pa if ha is None else ha + pa
        hb = pb if hb is None else hb + pb
    hbu = jnp.concatenate([hb[1:], hb[:1]], axis=0)
    h = jnp.maximum(ha + hbu + b1_ref[...], 0.0)
    h = jnp.dot(h.astype(jnp.bfloat16), w2_ref[...],
                preferred_element_type=jnp.float32)
    h = jnp.maximum(h + b2_ref[...], 0.0)
    o = jnp.dot(h.astype(jnp.bfloat16), w3_ref[...],
                preferred_element_type=jnp.float32)
    o_ref[...] = o + b3_ref[...]


def _run_branch(x, wb, cbmap, w1a, w1b, b1, w2, b2, w3, b3):
    n = x.shape[0]
    grid = (n // BS,)
    flops = 2 * n * (PIX * C_OUT * FRAME_PAD // 2
                     + C_OUT * FRAME_PAD * HID1 + HID1 * HID2 + HID2 * N_CLS)
    bytes_accessed = 4 * (n * PIX + n * N_CLS)
    return pl.pallas_call(
        _branch_kernel,
        out_shape=jax.ShapeDtypeStruct((n, N_CLS), jnp.float32),
        grid=grid,
        in_specs=[
            pl.BlockSpec((BS, PIX), lambda t: (t, 0)),
            pl.BlockSpec((PIX, C_OUT * FRAME_PAD), lambda t: (0, 0)),
            pl.BlockSpec((1, C_OUT * FRAME_PAD), lambda t: (0, 0)),
            pl.BlockSpec((C_OUT * FRAME_PAD, HID1), lambda t: (0, 0)),
            pl.BlockSpec((C_OUT * FRAME_PAD, HID1), lambda t: (0, 0)),
            pl.BlockSpec((1, HID1), lambda t: (0, 0)),
            pl.BlockSpec((HID1, HID2), lambda t: (0, 0)),
            pl.BlockSpec((1, HID2), lambda t: (0, 0)),
            pl.BlockSpec((HID2, N_CLS), lambda t: (0, 0)),
            pl.BlockSpec((1, N_CLS), lambda t: (0, 0)),
        ],
        out_specs=pl.BlockSpec((BS, N_CLS), lambda t: (t, 0)),
        compiler_params=pltpu.CompilerParams(
            dimension_semantics=("parallel",)),
        cost_estimate=pl.CostEstimate(flops=flops, transcendentals=0,
                                      bytes_accessed=bytes_accessed),
    )(x, wb, cbmap, w1a, w1b, b1, w2, b2, w3, b3)


def _pad_frame(t):
    pad = [(0, 0)] * (t.ndim - 1) + [(0, FRAME_PAD - FRAME)]
    return jnp.pad(t, pad).reshape(t.shape[:-2] + (C_OUT * FRAME_PAD,))


@functools.partial(jax.jit, static_argnames=())
def kernel(x1, x2, conv_w, conv_b, fc1_w, fc1_b, fc2_w, fc2_b, fc3_w, fc3_b):
    n = x1.shape[0]
    n_pad = (n + BS - 1) // BS * BS
    sel = _sel_mats()

    def fc1_half(w):
        w = w.reshape(HID1, C_OUT, H_POOL, W_POOL)
        w = jnp.pad(w, ((0, 0), (0, 0), (0, 1), (0, 1)))
        w = _pad_frame(w.reshape(HID1, C_OUT, FRAME))
        return w.T.astype(jnp.bfloat16)

    w1a = fc1_half(fc1_w[:, :FEAT_HALF])
    w1b = fc1_half(fc1_w[:, FEAT_HALF:])
    b1 = fc1_b.reshape(1, HID1)
    w2 = fc2_w.T.astype(jnp.bfloat16)
    b2 = fc2_b.reshape(1, HID2)
    w3 = fc3_w.T.astype(jnp.bfloat16)
    b3 = fc3_b.reshape(1, N_CLS)

    outs = []
    for b in range(2):
        taps = conv_w[b].reshape(C_OUT, KW, KW)
        wb = jnp.einsum('cde,dhi,ewj->hwcij', taps, sel, sel)
        wb = _pad_frame(wb.reshape(PIX, C_OUT, FRAME)).astype(jnp.bfloat16)
        cbmap = _pad_frame(
            jnp.repeat(conv_b[b], FRAME).reshape(C_OUT, FRAME)).reshape(1, -1)
        x = (x1 if b == 0 else x2).reshape(n, PIX)
        x = jnp.pad(x, ((0, n_pad - n), (0, 0)))
        o = _run_branch(x, wb, cbmap, w1a, w1b, b1, w2, b2, w3, b3)
        outs.append(o[0:n:2])
    return jnp.concatenate(outs, axis=0)


# speedup vs baseline: 16.3437x; 1.2331x over previous
import functools

import jax
import jax.numpy as jnp
import numpy as np
from jax.experimental import pallas as pl
from jax.experimental.pallas import tpu as pltpu

C_OUT = 5
KW = 3
H_IN, W_IN = 20, 20
PIX = H_IN * W_IN
H_CONV, W_CONV = H_IN - KW + 1, W_IN - KW + 1
H_POOL, W_POOL = H_CONV - 1, W_CONV - 1
FRAME = H_CONV * W_CONV
FRAME_PAD = 384
NL = C_OUT * FRAME_PAD
FEAT_HALF = C_OUT * H_POOL * W_POOL
HID1, HID2, N_CLS = 64, 16, 4
FC1N = 256
BS = 1024


def _index_rows():
    io = np.full((1, NL), -1000.0, np.float32)
    jo = np.full((1, NL), -1000.0, np.float32)
    ch = np.zeros((C_OUT, NL), np.float32)
    for c in range(C_OUT):
        ch[c, c * FRAME_PAD:(c + 1) * FRAME_PAD] = 1.0
        for i in range(H_CONV):
            for j in range(W_CONV):
                q = c * FRAME_PAD + i * W_CONV + j
                io[0, q] = i
                jo[0, q] = j
    hi = np.repeat(np.arange(H_IN), W_IN).astype(np.float32).reshape(PIX, 1)
    wi = np.tile(np.arange(W_IN), H_IN).astype(np.float32).reshape(PIX, 1)
    return (jnp.asarray(io), jnp.asarray(jo), jnp.asarray(ch),
            jnp.asarray(hi), jnp.asarray(wi))


def _prep_kernel(cw_ref, cb_in_ref, fc1T_ref, io_ref, jo_ref, ch_ref,
                 hi_ref, wi_ref, wb_ref, cb_ref, w1ab_ref):
    dh = hi_ref[...] - io_ref[...]
    dw = wi_ref[...] - jo_ref[...]
    for b in range(2):
        acc = jnp.zeros((PIX, NL), jnp.float32)
        for d in range(KW):
            for e in range(KW):
                tv = jnp.zeros((1, NL), jnp.float32)
                for c in range(C_OUT):
                    tv = tv + cw_ref[b, c * 9 + d * 3 + e] * ch_ref[c:c + 1, :]
                band = jnp.logical_and(dh == float(d), dw == float(e))
                acc = acc + jnp.where(band, tv, 0.0)
        wb_ref[b] = acc.astype(jnp.bfloat16)
        cbv = jnp.zeros((1, NL), jnp.float32)
        for c in range(C_OUT):
            cbv = cbv + cb_in_ref[b, c] * ch_ref[c:c + 1, :]
        cb_ref[b] = cbv
    w1ab_ref[...] = jnp.zeros((NL, FC1N), jnp.bfloat16)
    for c in range(C_OUT):
        for i in range(H_POOL):
            dst = c * FRAME_PAD + i * W_CONV
            src = c * H_POOL * W_POOL + i * W_POOL
            blk_a = fc1T_ref[src:src + W_POOL, :].astype(jnp.bfloat16)
            blk_b = fc1T_ref[FEAT_HALF + src:FEAT_HALF + src + W_POOL, :]
            w1ab_ref[dst:dst + W_POOL, 0:HID1] = blk_a
            w1ab_ref[dst:dst + W_POOL, 128:128 + HID1] = blk_b.astype(jnp.bfloat16)


def _prep(conv_w, conv_b, fc1T):
    io, jo, ch, hi, wi = _index_rows()
    return pl.pallas_call(
        _prep_kernel,
        out_shape=(
            jax.ShapeDtypeStruct((2, PIX, NL), jnp.bfloat16),
            jax.ShapeDtypeStruct((2, 1, NL), jnp.float32),
            jax.ShapeDtypeStruct((NL, FC1N), jnp.bfloat16),
        ),
        in_specs=[
            pl.BlockSpec(memory_space=pltpu.MemorySpace.SMEM),
            pl.BlockSpec(memory_space=pltpu.MemorySpace.SMEM),
            pl.BlockSpec((FEAT_HALF * 2, HID1), lambda: (0, 0)),
            pl.BlockSpec((1, NL), lambda: (0, 0)),
            pl.BlockSpec((1, NL), lambda: (0, 0)),
            pl.BlockSpec((C_OUT, NL), lambda: (0, 0)),
            pl.BlockSpec((PIX, 1), lambda: (0, 0)),
            pl.BlockSpec((PIX, 1), lambda: (0, 0)),
        ],
        out_specs=(
            pl.BlockSpec((2, PIX, NL), lambda: (0, 0, 0)),
            pl.BlockSpec((2, 1, NL), lambda: (0, 0, 0)),
            pl.BlockSpec((NL, FC1N), lambda: (0, 0)),
        ),
    )(conv_w, conv_b, fc1T, io, jo, ch, hi, wi)


def _branch_kernel(x_ref, wb_ref, cb_ref, w1ab_ref, b1_ref,
                   w2_ref, b2_ref, w3_ref, b3_ref, o_ref):
    x = x_ref[...].astype(jnp.bfloat16)
    hsum = None
    for c in range(C_OUT):
        lo = c * FRAME_PAD
        z = jnp.dot(x, wb_ref[:, lo:lo + FRAME_PAD],
                    preferred_element_type=jnp.float32)
        z = jnp.maximum(z + cb_ref[:, lo:lo + FRAME_PAD], 0.0)
        m = jnp.maximum(z, pltpu.roll(z, FRAME_PAD - 1, axis=1))
        m = jnp.maximum(m, pltpu.roll(m, FRAME_PAD - W_CONV, axis=1))
        p = jnp.dot(m.astype(jnp.bfloat16), w1ab_ref[lo:lo + FRAME_PAD, :],
                    preferred_element_type=jnp.float32)
        hsum = p if hsum is None else hsum + p
    ha = hsum[:, 0:HID1]
    hb = hsum[:, 128:128 + HID1]
    hbu = jnp.concatenate([hb[1:], hb[:1]], axis=0)
    h = jnp.maximum(ha + hbu + b1_ref[...], 0.0)
    h = jnp.dot(h.astype(jnp.bfloat16), w2_ref[...],
                preferred_element_type=jnp.float32)
    h = jnp.maximum(h + b2_ref[...], 0.0)
    o = jnp.dot(h.astype(jnp.bfloat16), w3_ref[...],
                preferred_element_type=jnp.float32)
    o_ref[...] = o + b3_ref[...]


def _run_branch(b, x, wb, cb, w1ab, b1, w2, b2, w3, b3):
    n = x.shape[0]
    grid = (n // BS,)
    flops = 2 * n * (PIX * NL // 2 + NL * FC1N // 2 + HID1 * HID2 + HID2 * N_CLS)
    bytes_accessed = 4 * (n * PIX + n * N_CLS)
    return pl.pallas_call(
        _branch_kernel,
        out_shape=jax.ShapeDtypeStruct((n, N_CLS), jnp.float32),
        grid=grid,
        in_specs=[
            pl.BlockSpec((BS, PIX), lambda t: (t, 0)),
            pl.BlockSpec((None, PIX, NL), lambda t, b=b: (b, 0, 0)),
            pl.BlockSpec((None, 1, NL), lambda t, b=b: (b, 0, 0)),
            pl.BlockSpec((NL, FC1N), lambda t: (0, 0)),
            pl.BlockSpec((1, HID1), lambda t: (0, 0)),
            pl.BlockSpec((HID1, HID2), lambda t: (0, 0)),
            pl.BlockSpec((1, HID2), lambda t: (0, 0)),
            pl.BlockSpec((HID2, N_CLS), lambda t: (0, 0)),
            pl.BlockSpec((1, N_CLS), lambda t: (0, 0)),
        ],
        out_specs=pl.BlockSpec((BS, N_CLS), lambda t: (t, 0)),
        compiler_params=pltpu.CompilerParams(
            dimension_semantics=("parallel",)),
        cost_estimate=pl.CostEstimate(flops=flops, transcendentals=0,
                                      bytes_accessed=bytes_accessed),
    )(x, wb, cb, w1ab, b1, w2, b2, w3, b3)


@functools.partial(jax.jit, static_argnames=())
def kernel(x1, x2, conv_w, conv_b, fc1_w, fc1_b, fc2_w, fc2_b, fc3_w, fc3_b):
    n = x1.shape[0]
    n_pad = (n + BS - 1) // BS * BS

    wb, cb, w1ab = _prep(conv_w, conv_b, fc1_w.T)
    b1 = fc1_b.reshape(1, HID1)
    w2 = fc2_w.T.astype(jnp.bfloat16)
    b2 = fc2_b.reshape(1, HID2)
    w3 = fc3_w.T.astype(jnp.bfloat16)
    b3 = fc3_b.reshape(1, N_CLS)

    outs = []
    for b in range(2):
        x = (x1 if b == 0 else x2).reshape(n, PIX)
        x = jnp.pad(x, ((0, n_pad - n), (0, 0)))
        o = _run_branch(b, x, wb, cb, w1ab, b1, w2, b2, w3, b3)
        outs.append(o[0:n:2])
    return jnp.concatenate(outs, axis=0)


# speedup vs baseline: 17.2664x; 1.0565x over previous
import functools

import jax
import jax.numpy as jnp
import numpy as np
from jax.experimental import pallas as pl
from jax.experimental.pallas import tpu as pltpu

C_OUT = 5
KW = 3
H_IN, W_IN = 20, 20
PIX = H_IN * W_IN
H_CONV, W_CONV = H_IN - KW + 1, W_IN - KW + 1
H_POOL, W_POOL = H_CONV - 1, W_CONV - 1
FRAME = H_CONV * W_CONV
FRAME_PAD = 384
NL = C_OUT * FRAME_PAD
FEAT_HALF = C_OUT * H_POOL * W_POOL
HID1, HID2, N_CLS = 64, 16, 4
FC1N = 256
BS = 1024


def _index_rows():
    io = np.full((1, NL), -1000.0, np.float32)
    jo = np.full((1, NL), -1000.0, np.float32)
    ch = np.zeros((C_OUT, NL), np.float32)
    for c in range(C_OUT):
        ch[c, c * FRAME_PAD:(c + 1) * FRAME_PAD] = 1.0
        for i in range(H_CONV):
            for j in range(W_CONV):
                q = c * FRAME_PAD + i * W_CONV + j
                io[0, q] = i
                jo[0, q] = j
    hi = np.repeat(np.arange(H_IN), W_IN).astype(np.float32).reshape(PIX, 1)
    wi = np.tile(np.arange(W_IN), H_IN).astype(np.float32).reshape(PIX, 1)
    return (jnp.asarray(io), jnp.asarray(jo), jnp.asarray(ch),
            jnp.asarray(hi), jnp.asarray(wi))


def _prep_kernel(cw_ref, cb_in_ref, fc1T_ref, io_ref, jo_ref, ch_ref,
                 hi_ref, wi_ref, wb_ref, cb_ref, w1ab_ref):
    dh = hi_ref[...] - io_ref[...]
    dw = wi_ref[...] - jo_ref[...]
    for b in range(2):
        acc = jnp.zeros((PIX, NL), jnp.float32)
        for d in range(KW):
            for e in range(KW):
                tv = jnp.zeros((1, NL), jnp.float32)
                for c in range(C_OUT):
                    tv = tv + cw_ref[b, c * 9 + d * 3 + e] * ch_ref[c:c + 1, :]
                band = jnp.logical_and(dh == float(d), dw == float(e))
                acc = acc + jnp.where(band, tv, 0.0)
        wb_ref[b] = acc.astype(jnp.bfloat16)
        cbv = jnp.zeros((1, NL), jnp.float32)
        for c in range(C_OUT):
            cbv = cbv + cb_in_ref[b, c] * ch_ref[c:c + 1, :]
        cb_ref[b] = cbv
    w1ab_ref[...] = jnp.zeros((NL, FC1N), jnp.bfloat16)
    for c in range(C_OUT):
        for i in range(H_POOL):
            dst = c * FRAME_PAD + i * W_CONV
            src = c * H_POOL * W_POOL + i * W_POOL
            blk_a = fc1T_ref[src:src + W_POOL, :].astype(jnp.bfloat16)
            blk_b = fc1T_ref[FEAT_HALF + src:FEAT_HALF + src + W_POOL, :]
            w1ab_ref[dst:dst + W_POOL, 0:HID1] = blk_a
            w1ab_ref[dst:dst + W_POOL, 128:128 + HID1] = blk_b.astype(jnp.bfloat16)


def _prep(conv_w, conv_b, fc1T):
    io, jo, ch, hi, wi = _index_rows()
    return pl.pallas_call(
        _prep_kernel,
        out_shape=(
            jax.ShapeDtypeStruct((2, PIX, NL), jnp.bfloat16),
            jax.ShapeDtypeStruct((2, 1, NL), jnp.float32),
            jax.ShapeDtypeStruct((NL, FC1N), jnp.bfloat16),
        ),
        in_specs=[
            pl.BlockSpec(memory_space=pltpu.MemorySpace.SMEM),
            pl.BlockSpec(memory_space=pltpu.MemorySpace.SMEM),
            pl.BlockSpec((FEAT_HALF * 2, HID1), lambda: (0, 0)),
            pl.BlockSpec((1, NL), lambda: (0, 0)),
            pl.BlockSpec((1, NL), lambda: (0, 0)),
            pl.BlockSpec((C_OUT, NL), lambda: (0, 0)),
            pl.BlockSpec((PIX, 1), lambda: (0, 0)),
            pl.BlockSpec((PIX, 1), lambda: (0, 0)),
        ],
        out_specs=(
            pl.BlockSpec((2, PIX, NL), lambda: (0, 0, 0)),
            pl.BlockSpec((2, 1, NL), lambda: (0, 0, 0)),
            pl.BlockSpec((NL, FC1N), lambda: (0, 0)),
        ),
    )(conv_w, conv_b, fc1T, io, jo, ch, hi, wi)


def _branch_kernel(x_ref, wb_ref, cb_ref, w1ab_ref, b1_ref,
                   w2_ref, b2_ref, w3_ref, b3_ref, o_ref):
    x = x_ref[...].astype(jnp.bfloat16)
    hsum = None
    for c in range(C_OUT):
        lo = c * FRAME_PAD
        z = jnp.dot(x, wb_ref[:, lo:lo + FRAME_PAD],
                    preferred_element_type=jnp.float32)
        z = jnp.maximum(z + cb_ref[:, lo:lo + FRAME_PAD], 0.0)
        m = jnp.maximum(z, pltpu.roll(z, FRAME_PAD - 1, axis=1))
        m = jnp.maximum(m, pltpu.roll(m, FRAME_PAD - W_CONV, axis=1))
        p = jnp.dot(m.astype(jnp.bfloat16), w1ab_ref[lo:lo + FRAME_PAD, :],
                    preferred_element_type=jnp.float32)
        hsum = p if hsum is None else hsum + p
    ha = hsum[:, 0:HID1]
    hb = hsum[:, 128:128 + HID1]
    hbu = jnp.concatenate([hb[1:], hb[:1]], axis=0)
    h = jnp.maximum(ha + hbu + b1_ref[...], 0.0)
    h = jnp.dot(h.astype(jnp.bfloat16), w2_ref[...],
                preferred_element_type=jnp.float32)
    h = jnp.maximum(h + b2_ref[...], 0.0)
    o = jnp.dot(h.astype(jnp.bfloat16), w3_ref[...],
                preferred_element_type=jnp.float32)
    o_ref[...] = o + b3_ref[...]


def _run_branch(b, x, wb, cb, w1ab, b1, w2, b2, w3, b3):
    n = x.shape[0]
    grid = (n // BS,)
    flops = 2 * n * (PIX * NL // 2 + NL * FC1N // 2 + HID1 * HID2 + HID2 * N_CLS)
    bytes_accessed = 4 * (n * PIX + n * N_CLS)
    return pl.pallas_call(
        _branch_kernel,
        out_shape=jax.ShapeDtypeStruct((n, N_CLS), jnp.float32),
        grid=grid,
        in_specs=[
            pl.BlockSpec((BS, PIX), lambda t: (t, 0)),
            pl.BlockSpec((None, PIX, NL), lambda t, b=b: (b, 0, 0)),
            pl.BlockSpec((None, 1, NL), lambda t, b=b: (b, 0, 0)),
            pl.BlockSpec((NL, FC1N), lambda t: (0, 0)),
            pl.BlockSpec((1, HID1), lambda t: (0, 0)),
            pl.BlockSpec((HID1, HID2), lambda t: (0, 0)),
            pl.BlockSpec((1, HID2), lambda t: (0, 0)),
            pl.BlockSpec((HID2, N_CLS), lambda t: (0, 0)),
            pl.BlockSpec((1, N_CLS), lambda t: (0, 0)),
        ],
        out_specs=pl.BlockSpec((BS, N_CLS), lambda t: (t, 0)),
        compiler_params=pltpu.CompilerParams(
            dimension_semantics=("parallel",)),
        cost_estimate=pl.CostEstimate(flops=flops, transcendentals=0,
                                      bytes_accessed=bytes_accessed),
    )(x, wb, cb, w1ab, b1, w2, b2, w3, b3)


@functools.partial(jax.jit, static_argnames=())
def kernel(x1, x2, conv_w, conv_b, fc1_w, fc1_b, fc2_w, fc2_b, fc3_w, fc3_b):
    n = x1.shape[0]
    n_pad = (n + BS - 1) // BS * BS

    wb, cb, w1ab = _prep(conv_w, conv_b, fc1_w.T)
    b1 = fc1_b.reshape(1, HID1)
    w2 = fc2_w.T.astype(jnp.bfloat16)
    b2 = fc2_b.reshape(1, HID2)
    w3 = fc3_w.T.astype(jnp.bfloat16)
    b3 = fc3_b.reshape(1, N_CLS)

    outs = []
    for b in range(2):
        x = (x1 if b == 0 else x2).reshape(n, PIX)
        x = jnp.pad(x, ((0, n_pad - n), (0, 0)))
        o = _run_branch(b, x, wb, cb, w1ab, b1, w2, b2, w3, b3)
        outs.append(o[:n].reshape(n // 2, 2 * N_CLS)[:, :N_CLS])
    return jnp.concatenate(outs, axis=0)


# speedup vs baseline: 17.8173x; 1.0319x over previous
import functools

import jax
import jax.numpy as jnp
import numpy as np
from jax.experimental import pallas as pl
from jax.experimental.pallas import tpu as pltpu

C_OUT = 5
KW = 3
H_IN, W_IN = 20, 20
PIX = H_IN * W_IN
H_CONV, W_CONV = H_IN - KW + 1, W_IN - KW + 1
H_POOL, W_POOL = H_CONV - 1, W_CONV - 1
FRAME = H_CONV * W_CONV
FRAME_PAD = 384
NL = C_OUT * FRAME_PAD
FEAT_HALF = C_OUT * H_POOL * W_POOL
HID1, HID2, N_CLS = 64, 16, 4
FC1N = 256
BS = 2048


def _index_rows():
    io = np.full((1, NL), -1000.0, np.float32)
    jo = np.full((1, NL), -1000.0, np.float32)
    ch = np.zeros((C_OUT, NL), np.float32)
    for c in range(C_OUT):
        ch[c, c * FRAME_PAD:(c + 1) * FRAME_PAD] = 1.0
        for i in range(H_CONV):
            for j in range(W_CONV):
                q = c * FRAME_PAD + i * W_CONV + j
                io[0, q] = i
                jo[0, q] = j
    hi = np.repeat(np.arange(H_IN), W_IN).astype(np.float32).reshape(PIX, 1)
    wi = np.tile(np.arange(W_IN), H_IN).astype(np.float32).reshape(PIX, 1)
    return (jnp.asarray(io), jnp.asarray(jo), jnp.asarray(ch),
            jnp.asarray(hi), jnp.asarray(wi))


def _prep_kernel(cw_ref, cb_in_ref, fc1T_ref, io_ref, jo_ref, ch_ref,
                 hi_ref, wi_ref, wb_ref, cb_ref, w1ab_ref):
    dh = hi_ref[...] - io_ref[...]
    dw = wi_ref[...] - jo_ref[...]
    for b in range(2):
        acc = jnp.zeros((PIX, NL), jnp.float32)
        for d in range(KW):
            for e in range(KW):
                tv = jnp.zeros((1, NL), jnp.float32)
                for c in range(C_OUT):
                    tv = tv + cw_ref[b, c * 9 + d * 3 + e] * ch_ref[c:c + 1, :]
                band = jnp.logical_and(dh == float(d), dw == float(e))
                acc = acc + jnp.where(band, tv, 0.0)
        wb_ref[b] = acc.astype(jnp.bfloat16)
        cbv = jnp.zeros((1, NL), jnp.float32)
        for c in range(C_OUT):
            cbv = cbv + cb_in_ref[b, c] * ch_ref[c:c + 1, :]
        cb_ref[b] = cbv
    w1ab_ref[...] = jnp.zeros((NL, FC1N), jnp.bfloat16)
    for c in range(C_OUT):
        for i in range(H_POOL):
            dst = c * FRAME_PAD + i * W_CONV
            src = c * H_POOL * W_POOL + i * W_POOL
            blk_a = fc1T_ref[src:src + W_POOL, :].astype(jnp.bfloat16)
            blk_b = fc1T_ref[FEAT_HALF + src:FEAT_HALF + src + W_POOL, :]
            w1ab_ref[dst:dst + W_POOL, 0:HID1] = blk_a
            w1ab_ref[dst:dst + W_POOL, 128:128 + HID1] = blk_b.astype(jnp.bfloat16)


def _prep(conv_w, conv_b, fc1T):
    io, jo, ch, hi, wi = _index_rows()
    return pl.pallas_call(
        _prep_kernel,
        out_shape=(
            jax.ShapeDtypeStruct((2, PIX, NL), jnp.bfloat16),
            jax.ShapeDtypeStruct((2, 1, NL), jnp.float32),
            jax.ShapeDtypeStruct((NL, FC1N), jnp.bfloat16),
        ),
        in_specs=[
            pl.BlockSpec(memory_space=pltpu.MemorySpace.SMEM),
            pl.BlockSpec(memory_space=pltpu.MemorySpace.SMEM),
            pl.BlockSpec((FEAT_HALF * 2, HID1), lambda: (0, 0)),
            pl.BlockSpec((1, NL), lambda: (0, 0)),
            pl.BlockSpec((1, NL), lambda: (0, 0)),
            pl.BlockSpec((C_OUT, NL), lambda: (0, 0)),
            pl.BlockSpec((PIX, 1), lambda: (0, 0)),
            pl.BlockSpec((PIX, 1), lambda: (0, 0)),
        ],
        out_specs=(
            pl.BlockSpec((2, PIX, NL), lambda: (0, 0, 0)),
            pl.BlockSpec((2, 1, NL), lambda: (0, 0, 0)),
            pl.BlockSpec((NL, FC1N), lambda: (0, 0)),
        ),
    )(conv_w, conv_b, fc1T, io, jo, ch, hi, wi)


def _branch_kernel(x_ref, wb_ref, cb_ref, w1ab_ref, b1_ref,
                   w2_ref, b2_ref, w3_ref, b3_ref, o_ref):
    x = x_ref[...]
    hsum = None
    for c in range(C_OUT):
        lo = c * FRAME_PAD
        z = jnp.dot(x, wb_ref[:, lo:lo + FRAME_PAD],
                    preferred_element_type=jnp.float32)
        z = jnp.maximum(z + cb_ref[:, lo:lo + FRAME_PAD], 0.0)
        m = jnp.maximum(z, pltpu.roll(z, FRAME_PAD - 1, axis=1))
        m = jnp.maximum(m, pltpu.roll(m, FRAME_PAD - W_CONV, axis=1))
        p = jnp.dot(m.astype(jnp.bfloat16), w1ab_ref[lo:lo + FRAME_PAD, :],
                    preferred_element_type=jnp.float32)
        hsum = p if hsum is None else hsum + p
    ha = hsum[:, 0:HID1]
    hb = hsum[:, 128:128 + HID1]
    hbu = jnp.concatenate([hb[1:], hb[:1]], axis=0)
    h = jnp.maximum(ha + hbu + b1_ref[...], 0.0)
    h = jnp.dot(h.astype(jnp.bfloat16), w2_ref[...],
                preferred_element_type=jnp.float32)
    h = jnp.maximum(h + b2_ref[...], 0.0)
    o = jnp.dot(h.astype(jnp.bfloat16), w3_ref[...],
                preferred_element_type=jnp.float32)
    o_ref[...] = o + b3_ref[...]


def _run_branch(b, x, wb, cb, w1ab, b1, w2, b2, w3, b3):
    n = x.shape[0]
    nt = n // BS
    half = max(nt // 2, 1)
    grid = (nt // half, half)
    flops = 2 * n * (PIX * NL // 2 + NL * FC1N // 2 + HID1 * HID2 + HID2 * N_CLS)
    bytes_accessed = 2 * n * PIX + 4 * n * N_CLS
    return pl.pallas_call(
        _branch_kernel,
        out_shape=jax.ShapeDtypeStruct((n, N_CLS), jnp.float32),
        grid=grid,
        in_specs=[
            pl.BlockSpec((BS, PIX), lambda s, t, h=half: (s * h + t, 0)),
            pl.BlockSpec((None, PIX, NL), lambda s, t, b=b: (b, 0, 0)),
            pl.BlockSpec((None, 1, NL), lambda s, t, b=b: (b, 0, 0)),
            pl.BlockSpec((NL, FC1N), lambda s, t: (0, 0)),
            pl.BlockSpec((1, HID1), lambda s, t: (0, 0)),
            pl.BlockSpec((HID1, HID2), lambda s, t: (0, 0)),
            pl.BlockSpec((1, HID2), lambda s, t: (0, 0)),
            pl.BlockSpec((HID2, N_CLS), lambda s, t: (0, 0)),
            pl.BlockSpec((1, N_CLS), lambda s, t: (0, 0)),
        ],
        out_specs=pl.BlockSpec((BS, N_CLS), lambda s, t, h=half: (s * h + t, 0)),
        compiler_params=pltpu.CompilerParams(
            dimension_semantics=("parallel", "parallel")),
        cost_estimate=pl.CostEstimate(flops=flops, transcendentals=0,
                                      bytes_accessed=bytes_accessed),
    )(x, wb, cb, w1ab, b1, w2, b2, w3, b3)


@functools.partial(jax.jit, static_argnames=())
def kernel(x1, x2, conv_w, conv_b, fc1_w, fc1_b, fc2_w, fc2_b, fc3_w, fc3_b):
    n = x1.shape[0]
    n_pad = (n + BS - 1) // BS * BS

    wb, cb, w1ab = _prep(conv_w, conv_b, fc1_w.T)
    b1 = fc1_b.reshape(1, HID1)
    w2 = fc2_w.T.astype(jnp.bfloat16)
    b2 = fc2_b.reshape(1, HID2)
    w3 = fc3_w.T.astype(jnp.bfloat16)
    b3 = fc3_b.reshape(1, N_CLS)

    outs = []
    for b in range(2):
        x = (x1 if b == 0 else x2).reshape(n, PIX).astype(jnp.bfloat16)
        if n_pad != n:
            x = jnp.pad(x, ((0, n_pad - n), (0, 0)))
        o = _run_branch(b, x, wb, cb, w1ab, b1, w2, b2, w3, b3)
        outs.append(o[:n].reshape(n // 2, 2 * N_CLS)[:, :N_CLS])
    return jnp.concatenate(outs, axis=0)


# speedup vs baseline: 19.8493x; 1.1140x over previous
import functools

import jax
import jax.numpy as jnp
import numpy as np
from jax.experimental import pallas as pl
from jax.experimental.pallas import tpu as pltpu

C_OUT = 5
KW = 3
H_IN, W_IN = 20, 20
PIX = H_IN * W_IN
H_CONV, W_CONV = H_IN - KW + 1, W_IN - KW + 1
H_POOL, W_POOL = H_CONV - 1, W_CONV - 1
FRAME = H_CONV * W_CONV
FRAME_PAD = 384
NL = C_OUT * FRAME_PAD
FEAT_HALF = C_OUT * H_POOL * W_POOL
HID1, HID2, N_CLS = 64, 16, 4
FC1N = 256
BS = 4096


def _index_rows():
    io = np.full((1, NL), -1000.0, np.float32)
    jo = np.full((1, NL), -1000.0, np.float32)
    ch = np.zeros((C_OUT, NL), np.float32)
    for c in range(C_OUT):
        ch[c, c * FRAME_PAD:(c + 1) * FRAME_PAD] = 1.0
        for i in range(H_CONV):
            for j in range(W_CONV):
                q = c * FRAME_PAD + i * W_CONV + j
                io[0, q] = i
                jo[0, q] = j
    hi = np.repeat(np.arange(H_IN), W_IN).astype(np.float32).reshape(PIX, 1)
    wi = np.tile(np.arange(W_IN), H_IN).astype(np.float32).reshape(PIX, 1)
    return (jnp.asarray(io), jnp.asarray(jo), jnp.asarray(ch),
            jnp.asarray(hi), jnp.asarray(wi))


def _prep_kernel(cw_ref, cb_in_ref, fc1T_ref, io_ref, jo_ref, ch_ref,
                 hi_ref, wi_ref, wb_ref, cb_ref, w1ab_ref):
    dh = hi_ref[...] - io_ref[...]
    dw = wi_ref[...] - jo_ref[...]
    for b in range(2):
        acc = jnp.zeros((PIX, NL), jnp.float32)
        for d in range(KW):
            for e in range(KW):
                tv = jnp.zeros((1, NL), jnp.float32)
                for c in range(C_OUT):
                    tv = tv + cw_ref[b, c * 9 + d * 3 + e] * ch_ref[c:c + 1, :]
                band = jnp.logical_and(dh == float(d), dw == float(e))
                acc = acc + jnp.where(band, tv, 0.0)
        wb_ref[b] = acc.astype(jnp.bfloat16)
        cbv = jnp.zeros((1, NL), jnp.float32)
        for c in range(C_OUT):
            cbv = cbv + cb_in_ref[b, c] * ch_ref[c:c + 1, :]
        cb_ref[b] = cbv
    w1ab_ref[...] = jnp.zeros((NL, FC1N), jnp.bfloat16)
    for c in range(C_OUT):
        for i in range(H_POOL):
            dst = c * FRAME_PAD + i * W_CONV
            src = c * H_POOL * W_POOL + i * W_POOL
            blk_a = fc1T_ref[src:src + W_POOL, :].astype(jnp.bfloat16)
            blk_b = fc1T_ref[FEAT_HALF + src:FEAT_HALF + src + W_POOL, :]
            w1ab_ref[dst:dst + W_POOL, 0:HID1] = blk_a
            w1ab_ref[dst:dst + W_POOL, 128:128 + HID1] = blk_b.astype(jnp.bfloat16)


def _prep(conv_w, conv_b, fc1T):
    io, jo, ch, hi, wi = _index_rows()
    return pl.pallas_call(
        _prep_kernel,
        out_shape=(
            jax.ShapeDtypeStruct((2, PIX, NL), jnp.bfloat16),
            jax.ShapeDtypeStruct((2, 1, NL), jnp.float32),
            jax.ShapeDtypeStruct((NL, FC1N), jnp.bfloat16),
        ),
        in_specs=[
            pl.BlockSpec(memory_space=pltpu.MemorySpace.SMEM),
            pl.BlockSpec(memory_space=pltpu.MemorySpace.SMEM),
            pl.BlockSpec((FEAT_HALF * 2, HID1), lambda: (0, 0)),
            pl.BlockSpec((1, NL), lambda: (0, 0)),
            pl.BlockSpec((1, NL), lambda: (0, 0)),
            pl.BlockSpec((C_OUT, NL), lambda: (0, 0)),
            pl.BlockSpec((PIX, 1), lambda: (0, 0)),
            pl.BlockSpec((PIX, 1), lambda: (0, 0)),
        ],
        out_specs=(
            pl.BlockSpec((2, PIX, NL), lambda: (0, 0, 0)),
            pl.BlockSpec((2, 1, NL), lambda: (0, 0, 0)),
            pl.BlockSpec((NL, FC1N), lambda: (0, 0)),
        ),
    )(conv_w, conv_b, fc1T, io, jo, ch, hi, wi)


def _branch_kernel(x_ref, wb_ref, cb_ref, w1ab_ref, b1_ref,
                   w2_ref, b2_ref, w3_ref, b3_ref, o_ref):
    x = x_ref[...]
    hsum = None
    for c in range(C_OUT):
        lo = c * FRAME_PAD
        z = jnp.dot(x, wb_ref[:, lo:lo + FRAME_PAD],
                    preferred_element_type=jnp.float32)
        z = jnp.maximum(z + cb_ref[:, lo:lo + FRAME_PAD], 0.0)
        zb = z.astype(jnp.bfloat16)
        m = jnp.maximum(zb, jnp.concatenate(
            [zb[:, 1:], zb[:, :1]], axis=1))
        m = jnp.maximum(m, jnp.concatenate(
            [m[:, W_CONV:], m[:, :W_CONV]], axis=1))
        p = jnp.dot(m, w1ab_ref[lo:lo + FRAME_PAD, :],
                    preferred_element_type=jnp.float32)
        hsum = p if hsum is None else hsum + p
    ha = hsum[:, 0:HID1]
    hb = hsum[:, 128:128 + HID1]
    hbu = jnp.concatenate([hb[1:], hb[:1]], axis=0)
    h = jnp.maximum(ha + hbu + b1_ref[...], 0.0)
    h = jnp.dot(h.astype(jnp.bfloat16), w2_ref[...],
                preferred_element_type=jnp.float32)
    h = jnp.maximum(h + b2_ref[...], 0.0)
    o = jnp.dot(h.astype(jnp.bfloat16), w3_ref[...],
                preferred_element_type=jnp.float32)
    o_ref[...] = o + b3_ref[...]


def _run_branch(b, x, wb, cb, w1ab, b1, w2, b2, w3, b3):
    n = x.shape[0]
    nt = n // BS
    half = max(nt // 2, 1)
    grid = (nt // half, half)
    flops = 2 * n * (PIX * NL // 2 + NL * FC1N // 2 + HID1 * HID2 + HID2 * N_CLS)
    bytes_accessed = 2 * n * PIX + 4 * n * N_CLS
    return pl.pallas_call(
        _branch_kernel,
        out_shape=jax.ShapeDtypeStruct((n, N_CLS), jnp.float32),
        grid=grid,
        in_specs=[
            pl.BlockSpec((BS, PIX), lambda s, t, h=half: (s * h + t, 0)),
            pl.BlockSpec((None, PIX, NL), lambda s, t, b=b: (b, 0, 0)),
            pl.BlockSpec((None, 1, NL), lambda s, t, b=b: (b, 0, 0)),
            pl.BlockSpec((NL, FC1N), lambda s, t: (0, 0)),
            pl.BlockSpec((1, HID1), lambda s, t: (0, 0)),
            pl.BlockSpec((HID1, HID2), lambda s, t: (0, 0)),
            pl.BlockSpec((1, HID2), lambda s, t: (0, 0)),
            pl.BlockSpec((HID2, N_CLS), lambda s, t: (0, 0)),
            pl.BlockSpec((1, N_CLS), lambda s, t: (0, 0)),
        ],
        out_specs=pl.BlockSpec((BS, N_CLS), lambda s, t, h=half: (s * h + t, 0)),
        compiler_params=pltpu.CompilerParams(
            dimension_semantics=("parallel", "parallel")),
        cost_estimate=pl.CostEstimate(flops=flops, transcendentals=0,
                                      bytes_accessed=bytes_accessed),
    )(x, wb, cb, w1ab, b1, w2, b2, w3, b3)


@functools.partial(jax.jit, static_argnames=())
def kernel(x1, x2, conv_w, conv_b, fc1_w, fc1_b, fc2_w, fc2_b, fc3_w, fc3_b):
    n = x1.shape[0]
    n_pad = (n + BS - 1) // BS * BS

    wb, cb, w1ab = _prep(conv_w, conv_b, fc1_w.T)
    b1 = fc1_b.reshape(1, HID1)
    w2 = fc2_w.T.astype(jnp.bfloat16)
    b2 = fc2_b.reshape(1, HID2)
    w3 = fc3_w.T.astype(jnp.bfloat16)
    b3 = fc3_b.reshape(1, N_CLS)

    outs = []
    for b in range(2):
        x = (x1 if b == 0 else x2).reshape(n, PIX).astype(jnp.bfloat16)
        if n_pad != n:
            x = jnp.pad(x, ((0, n_pad - n), (0, 0)))
        o = _run_branch(b, x, wb, cb, w1ab, b1, w2, b2, w3, b3)
        outs.append(o[:n].reshape(n // 2, 2 * N_CLS)[:, :N_CLS])
    return jnp.concatenate(outs, axis=0)


# speedup vs baseline: 19.9254x; 1.0038x over previous
import functools

import jax
import jax.numpy as jnp
import numpy as np
from jax.experimental import pallas as pl
from jax.experimental.pallas import tpu as pltpu

C_OUT = 5
KW = 3
H_IN, W_IN = 20, 20
PIX = H_IN * W_IN
H_CONV, W_CONV = H_IN - KW + 1, W_IN - KW + 1
H_POOL, W_POOL = H_CONV - 1, W_CONV - 1
FRAME = H_CONV * W_CONV
FRAME_PAD = 384
NL = C_OUT * FRAME_PAD
FEAT_HALF = C_OUT * H_POOL * W_POOL
HID1, HID2, N_CLS = 64, 16, 4
FC1N = 256
BS = 4096


def _index_rows():
    io = np.full((1, NL), -1000.0, np.float32)
    jo = np.full((1, NL), -1000.0, np.float32)
    ch = np.zeros((C_OUT, NL), np.float32)
    for c in range(C_OUT):
        ch[c, c * FRAME_PAD:(c + 1) * FRAME_PAD] = 1.0
        for i in range(H_CONV):
            for j in range(W_CONV):
                q = c * FRAME_PAD + i * W_CONV + j
                io[0, q] = i
                jo[0, q] = j
    hi = np.repeat(np.arange(H_IN), W_IN).astype(np.float32).reshape(PIX, 1)
    wi = np.tile(np.arange(W_IN), H_IN).astype(np.float32).reshape(PIX, 1)
    return (jnp.asarray(io), jnp.asarray(jo), jnp.asarray(ch),
            jnp.asarray(hi), jnp.asarray(wi))


def _prep_kernel(cw_ref, cb_in_ref, fc1T_ref, io_ref, jo_ref, ch_ref,
                 hi_ref, wi_ref, wb_ref, cb_ref, w1ab_ref):
    dh = hi_ref[...] - io_ref[...]
    dw = wi_ref[...] - jo_ref[...]
    for b in range(2):
        acc = jnp.zeros((PIX, NL), jnp.float32)
        for d in range(KW):
            for e in range(KW):
                tv = jnp.zeros((1, NL), jnp.float32)
                for c in range(C_OUT):
                    tv = tv + cw_ref[b, c * 9 + d * 3 + e] * ch_ref[c:c + 1, :]
                band = jnp.logical_and(dh == float(d), dw == float(e))
                acc = acc + jnp.where(band, tv, 0.0)
        wb_ref[b] = acc.astype(jnp.bfloat16)
        cbv = jnp.zeros((1, NL), jnp.float32)
        for c in range(C_OUT):
            cbv = cbv + cb_in_ref[b, c] * ch_ref[c:c + 1, :]
        cb_ref[b] = cbv
    w1ab_ref[...] = jnp.zeros((NL, FC1N), jnp.bfloat16)
    for c in range(C_OUT):
        for i in range(H_POOL):
            dst = c * FRAME_PAD + i * W_CONV
            src = c * H_POOL * W_POOL + i * W_POOL
            blk_a = fc1T_ref[src:src + W_POOL, :].astype(jnp.bfloat16)
            blk_b = fc1T_ref[FEAT_HALF + src:FEAT_HALF + src + W_POOL, :]
            w1ab_ref[dst:dst + W_POOL, 0:HID1] = blk_a
            w1ab_ref[dst:dst + W_POOL, 128:128 + HID1] = blk_b.astype(jnp.bfloat16)


def _prep(conv_w, conv_b, fc1T):
    io, jo, ch, hi, wi = _index_rows()
    return pl.pallas_call(
        _prep_kernel,
        out_shape=(
            jax.ShapeDtypeStruct((2, PIX, NL), jnp.bfloat16),
            jax.ShapeDtypeStruct((2, 1, NL), jnp.float32),
            jax.ShapeDtypeStruct((NL, FC1N), jnp.bfloat16),
        ),
        in_specs=[
            pl.BlockSpec(memory_space=pltpu.MemorySpace.SMEM),
            pl.BlockSpec(memory_space=pltpu.MemorySpace.SMEM),
            pl.BlockSpec((FEAT_HALF * 2, HID1), lambda: (0, 0)),
            pl.BlockSpec((1, NL), lambda: (0, 0)),
            pl.BlockSpec((1, NL), lambda: (0, 0)),
            pl.BlockSpec((C_OUT, NL), lambda: (0, 0)),
            pl.BlockSpec((PIX, 1), lambda: (0, 0)),
            pl.BlockSpec((PIX, 1), lambda: (0, 0)),
        ],
        out_specs=(
            pl.BlockSpec((2, PIX, NL), lambda: (0, 0, 0)),
            pl.BlockSpec((2, 1, NL), lambda: (0, 0, 0)),
            pl.BlockSpec((NL, FC1N), lambda: (0, 0)),
        ),
    )(conv_w, conv_b, fc1T, io, jo, ch, hi, wi)


def _branch_kernel(x_ref, wb_ref, cb_ref, w1ab_ref, b1_ref,
                   w2_ref, b2_ref, w3_ref, b3_ref, o_ref):
    x = x_ref[...]
    feats = []
    for c in range(C_OUT):
        lo = c * FRAME_PAD
        z = jnp.dot(x, wb_ref[:, lo:lo + FRAME_PAD],
                    preferred_element_type=jnp.float32)
        z = jnp.maximum(z + cb_ref[:, lo:lo + FRAME_PAD], 0.0)
        zb = z.astype(jnp.bfloat16)
        m = jnp.maximum(zb, jnp.concatenate(
            [zb[:, 1:], zb[:, :1]], axis=1))
        m = jnp.maximum(m, jnp.concatenate(
            [m[:, W_CONV:], m[:, :W_CONV]], axis=1))
        feats.append(m)
    hsum = jnp.dot(jnp.concatenate(feats, axis=1), w1ab_ref[...],
                   preferred_element_type=jnp.float32)
    ha = hsum[:, 0:HID1]
    hb = hsum[:, 128:128 + HID1]
    hbu = jnp.concatenate([hb[1:], hb[:1]], axis=0)
    h = jnp.maximum(ha + hbu + b1_ref[...], 0.0)
    h = jnp.dot(h.astype(jnp.bfloat16), w2_ref[...],
                preferred_element_type=jnp.float32)
    h = jnp.maximum(h + b2_ref[...], 0.0)
    o = jnp.dot(h.astype(jnp.bfloat16), w3_ref[...],
                preferred_element_type=jnp.float32)
    o_ref[...] = o + b3_ref[...]


def _run_branch(b, x, wb, cb, w1ab, b1, w2, b2, w3, b3):
    n = x.shape[0]
    nt = n // BS
    half = max(nt // 2, 1)
    grid = (nt // half, half)
    flops = 2 * n * (PIX * NL // 2 + NL * FC1N // 2 + HID1 * HID2 + HID2 * N_CLS)
    bytes_accessed = 2 * n * PIX + 4 * n * N_CLS
    return pl.pallas_call(
        _branch_kernel,
        out_shape=jax.ShapeDtypeStruct((n, N_CLS), jnp.float32),
        grid=grid,
        in_specs=[
            pl.BlockSpec((BS, PIX), lambda s, t, h=half: (s * h + t, 0)),
            pl.BlockSpec((None, PIX, NL), lambda s, t, b=b: (b, 0, 0)),
            pl.BlockSpec((None, 1, NL), lambda s, t, b=b: (b, 0, 0)),
            pl.BlockSpec((NL, FC1N), lambda s, t: (0, 0)),
            pl.BlockSpec((1, HID1), lambda s, t: (0, 0)),
            pl.BlockSpec((HID1, HID2), lambda s, t: (0, 0)),
            pl.BlockSpec((1, HID2), lambda s, t: (0, 0)),
            pl.BlockSpec((HID2, N_CLS), lambda s, t: (0, 0)),
            pl.BlockSpec((1, N_CLS), lambda s, t: (0, 0)),
        ],
        out_specs=pl.BlockSpec((BS, N_CLS), lambda s, t, h=half: (s * h + t, 0)),
        compiler_params=pltpu.CompilerParams(
            dimension_semantics=("parallel", "parallel")),
        cost_estimate=pl.CostEstimate(flops=flops, transcendentals=0,
                                      bytes_accessed=bytes_accessed),
    )(x, wb, cb, w1ab, b1, w2, b2, w3, b3)


@functools.partial(jax.jit, static_argnames=())
def kernel(x1, x2, conv_w, conv_b, fc1_w, fc1_b, fc2_w, fc2_b, fc3_w, fc3_b):
    n = x1.shape[0]
    n_pad = (n + BS - 1) // BS * BS

    wb, cb, w1ab = _prep(conv_w, conv_b, fc1_w.T)
    b1 = fc1_b.reshape(1, HID1)
    w2 = fc2_w.T.astype(jnp.bfloat16)
    b2 = fc2_b.reshape(1, HID2)
    w3 = fc3_w.T.astype(jnp.bfloat16)
    b3 = fc3_b.reshape(1, N_CLS)

    outs = []
    for b in range(2):
        x = (x1 if b == 0 else x2).reshape(n, PIX).astype(jnp.bfloat16)
        if n_pad != n:
            x = jnp.pad(x, ((0, n_pad - n), (0, 0)))
        o = _run_branch(b, x, wb, cb, w1ab, b1, w2, b2, w3, b3)
        outs.append(o[:n].reshape(n // 2, 2 * N_CLS)[:, :N_CLS])
    return jnp.concatenate(outs, axis=0)


# speedup vs baseline: 19.9286x; 1.0002x over previous
import functools

import jax
import jax.numpy as jnp
import numpy as np
from jax.experimental import pallas as pl
from jax.experimental.pallas import tpu as pltpu

C_OUT = 5
KW = 3
H_IN, W_IN = 20, 20
PIX = H_IN * W_IN
H_CONV, W_CONV = H_IN - KW + 1, W_IN - KW + 1
H_POOL, W_POOL = H_CONV - 1, W_CONV - 1
FRAME = H_CONV * W_CONV
FRAME_PAD = 384
NL = C_OUT * FRAME_PAD
FEAT_HALF = C_OUT * H_POOL * W_POOL
HID1, HID2, N_CLS = 64, 16, 4
FC1N = 256
BS = 4096


def _index_rows():
    io = np.full((1, NL), -1000.0, np.float32)
    jo = np.full((1, NL), -1000.0, np.float32)
    ch = np.zeros((C_OUT, NL), np.float32)
    for c in range(C_OUT):
        ch[c, c * FRAME_PAD:(c + 1) * FRAME_PAD] = 1.0
        for i in range(H_CONV):
            for j in range(W_CONV):
                q = c * FRAME_PAD + i * W_CONV + j
                io[0, q] = i
                jo[0, q] = j
    hi = np.repeat(np.arange(H_IN), W_IN).astype(np.float32).reshape(PIX, 1)
    wi = np.tile(np.arange(W_IN), H_IN).astype(np.float32).reshape(PIX, 1)
    return (jnp.asarray(io), jnp.asarray(jo), jnp.asarray(ch),
            jnp.asarray(hi), jnp.asarray(wi))


def _prep_kernel(cw_ref, cb_in_ref, fc1T_ref, io_ref, jo_ref, ch_ref,
                 hi_ref, wi_ref, wb_ref, cb_ref, w1ab_ref):
    dh = hi_ref[...] - io_ref[...]
    dw = wi_ref[...] - jo_ref[...]
    for b in range(2):
        acc = jnp.zeros((PIX, NL), jnp.float32)
        for d in range(KW):
            for e in range(KW):
                tv = jnp.zeros((1, NL), jnp.float32)
                for c in range(C_OUT):
                    tv = tv + cw_ref[b, c * 9 + d * 3 + e] * ch_ref[c:c + 1, :]
                band = jnp.logical_and(dh == float(d), dw == float(e))
                acc = acc + jnp.where(band, tv, 0.0)
        wb_ref[b] = acc.astype(jnp.bfloat16)
        cbv = jnp.zeros((1, NL), jnp.float32)
        for c in range(C_OUT):
            cbv = cbv + cb_in_ref[b, c] * ch_ref[c:c + 1, :]
        cb_ref[b] = cbv
    w1ab_ref[...] = jnp.zeros((NL, FC1N), jnp.bfloat16)
    for c in range(C_OUT):
        for i in range(H_POOL):
            dst = c * FRAME_PAD + i * W_CONV
            src = c * H_POOL * W_POOL + i * W_POOL
            blk_a = fc1T_ref[src:src + W_POOL, :].astype(jnp.bfloat16)
            blk_b = fc1T_ref[FEAT_HALF + src:FEAT_HALF + src + W_POOL, :]
            w1ab_ref[dst:dst + W_POOL, 0:HID1] = blk_a
            w1ab_ref[dst:dst + W_POOL, 128:128 + HID1] = blk_b.astype(jnp.bfloat16)


def _prep(conv_w, conv_b, fc1T):
    io, jo, ch, hi, wi = _index_rows()
    return pl.pallas_call(
        _prep_kernel,
        out_shape=(
            jax.ShapeDtypeStruct((2, PIX, NL), jnp.bfloat16),
            jax.ShapeDtypeStruct((2, 1, NL), jnp.float32),
            jax.ShapeDtypeStruct((NL, FC1N), jnp.bfloat16),
        ),
        in_specs=[
            pl.BlockSpec(memory_space=pltpu.MemorySpace.SMEM),
            pl.BlockSpec(memory_space=pltpu.MemorySpace.SMEM),
            pl.BlockSpec((FEAT_HALF * 2, HID1), lambda: (0, 0)),
            pl.BlockSpec((1, NL), lambda: (0, 0)),
            pl.BlockSpec((1, NL), lambda: (0, 0)),
            pl.BlockSpec((C_OUT, NL), lambda: (0, 0)),
            pl.BlockSpec((PIX, 1), lambda: (0, 0)),
            pl.BlockSpec((PIX, 1), lambda: (0, 0)),
        ],
        out_specs=(
            pl.BlockSpec((2, PIX, NL), lambda: (0, 0, 0)),
            pl.BlockSpec((2, 1, NL), lambda: (0, 0, 0)),
            pl.BlockSpec((NL, FC1N), lambda: (0, 0)),
        ),
    )(conv_w, conv_b, fc1T, io, jo, ch, hi, wi)


def _branch_kernel(x_ref, wb_ref, cb_ref, w1ab_ref, b1_ref,
                   w2_ref, b2_ref, w3_ref, b3_ref, o_ref):
    x = x_ref[...]
    feats = []
    for c in range(C_OUT):
        lo = c * FRAME_PAD
        z = jnp.dot(x, wb_ref[:, lo:lo + FRAME_PAD],
                    preferred_element_type=jnp.float32)
        z = jnp.maximum(z + cb_ref[:, lo:lo + FRAME_PAD], 0.0)
        zb = z.astype(jnp.bfloat16)
        m = jnp.maximum(zb, jnp.concatenate(
            [zb[:, 1:], zb[:, :1]], axis=1))
        m = jnp.maximum(m, jnp.concatenate(
            [m[:, W_CONV:], m[:, :W_CONV]], axis=1))
        feats.append(m)
    hsum = jnp.dot(jnp.concatenate(feats, axis=1), w1ab_ref[...],
                   preferred_element_type=jnp.float32)
    ha = hsum[:, 0:HID1]
    hb = hsum[:, 128:128 + HID1]
    hbu = jnp.concatenate([hb[1:], hb[:1]], axis=0)
    h = jnp.maximum(ha + hbu + b1_ref[...], 0.0)
    h = jnp.dot(h.astype(jnp.bfloat16), w2_ref[...],
                preferred_element_type=jnp.float32)
    h = jnp.maximum(h + b2_ref[...], 0.0)
    o = jnp.dot(h.astype(jnp.bfloat16), w3_ref[...],
                preferred_element_type=jnp.float32)
    o_ref[...] = o + b3_ref[...]


def _run_branch(b, x, wb, cb, w1ab, b1, w2, b2, w3, b3):
    n = x.shape[0]
    nt = n // BS
    half = max(nt // 2, 1)
    grid = (nt // half, half)
    flops = 2 * n * (PIX * NL // 2 + NL * FC1N // 2 + HID1 * HID2 + HID2 * N_CLS)
    bytes_accessed = 2 * n * PIX + 4 * n * N_CLS
    return pl.pallas_call(
        _branch_kernel,
        out_shape=jax.ShapeDtypeStruct((n, N_CLS), jnp.float32),
        grid=grid,
        in_specs=[
            pl.BlockSpec((BS, PIX), lambda s, t, h=half: (s * h + t, 0)),
            pl.BlockSpec((None, PIX, NL), lambda s, t, b=b: (b, 0, 0)),
            pl.BlockSpec((None, 1, NL), lambda s, t, b=b: (b, 0, 0)),
            pl.BlockSpec((NL, FC1N), lambda s, t: (0, 0)),
            pl.BlockSpec((1, HID1), lambda s, t: (0, 0)),
            pl.BlockSpec((HID1, HID2), lambda s, t: (0, 0)),
            pl.BlockSpec((1, HID2), lambda s, t: (0, 0)),
            pl.BlockSpec((HID2, N_CLS), lambda s, t: (0, 0)),
            pl.BlockSpec((1, N_CLS), lambda s, t: (0, 0)),
        ],
        out_specs=pl.BlockSpec((BS, N_CLS), lambda s, t, h=half: (s * h + t, 0)),
        compiler_params=pltpu.CompilerParams(
            dimension_semantics=("parallel", "parallel")),
        cost_estimate=pl.CostEstimate(flops=flops, transcendentals=0,
                                      bytes_accessed=bytes_accessed),
    )(x, wb, cb, w1ab, b1, w2, b2, w3, b3)


@functools.partial(jax.jit, static_argnames=())
def kernel(x1, x2, conv_w, conv_b, fc1_w, fc1_b, fc2_w, fc2_b, fc3_w, fc3_b):
    n = x1.shape[0]
    n_pad = (n + BS - 1) // BS * BS

    wb, cb, w1ab = _prep(conv_w, conv_b, fc1_w.T)
    b1 = fc1_b.reshape(1, HID1)
    w2 = fc2_w.T.astype(jnp.bfloat16)
    b2 = fc2_b.reshape(1, HID2)
    w3 = fc3_w.T.astype(jnp.bfloat16)
    b3 = fc3_b.reshape(1, N_CLS)

    outs = []
    for b in range(2):
        x = (x1 if b == 0 else x2).astype(jnp.bfloat16).reshape(n, PIX)
        if n_pad != n:
            x = jnp.pad(x, ((0, n_pad - n), (0, 0)))
        o = _run_branch(b, x, wb, cb, w1ab, b1, w2, b2, w3, b3)
        outs.append(o[:n].reshape(n // 2, 2 * N_CLS)[:, :N_CLS])
    return jnp.concatenate(outs, axis=0)


# speedup vs baseline: 21.2720x; 1.0674x over previous
import functools

import jax
import jax.numpy as jnp
import numpy as np
from jax.experimental import pallas as pl
from jax.experimental.pallas import tpu as pltpu

C_OUT = 5
KW = 3
H_IN, W_IN = 20, 20
PIX = H_IN * W_IN
H_CONV, W_CONV = H_IN - KW + 1, W_IN - KW + 1
H_POOL, W_POOL = H_CONV - 1, W_CONV - 1
FRAME = H_CONV * W_CONV
NL = 1664
FEAT_HALF = C_OUT * H_POOL * W_POOL
HID1, HID2, N_CLS = 64, 16, 4
FC1N = 256
BS = 4096


def _index_rows():
    io = np.full((1, NL), -1000.0, np.float32)
    jo = np.full((1, NL), -1000.0, np.float32)
    ch = np.zeros((C_OUT, NL), np.float32)
    for c in range(C_OUT):
        ch[c, c * FRAME:(c + 1) * FRAME] = 1.0
        for i in range(H_CONV):
            for j in range(W_CONV):
                q = c * FRAME + i * W_CONV + j
                io[0, q] = i
                jo[0, q] = j
    hi = np.repeat(np.arange(H_IN), W_IN).astype(np.float32).reshape(PIX, 1)
    wi = np.tile(np.arange(W_IN), H_IN).astype(np.float32).reshape(PIX, 1)
    return (jnp.asarray(io), jnp.asarray(jo), jnp.asarray(ch),
            jnp.asarray(hi), jnp.asarray(wi))


def _prep_kernel(cw_ref, cb_in_ref, fc1T_ref, io_ref, jo_ref, ch_ref,
                 hi_ref, wi_ref, wb_ref, cb_ref, w1ab_ref):
    dh = hi_ref[...] - io_ref[...]
    dw = wi_ref[...] - jo_ref[...]
    for b in range(2):
        acc = jnp.zeros((PIX, NL), jnp.float32)
        for d in range(KW):
            for e in range(KW):
                tv = jnp.zeros((1, NL), jnp.float32)
                for c in range(C_OUT):
                    tv = tv + cw_ref[b, c * 9 + d * 3 + e] * ch_ref[c:c + 1, :]
                band = jnp.logical_and(dh == float(d), dw == float(e))
                acc = acc + jnp.where(band, tv, 0.0)
        wb_ref[b] = acc.astype(jnp.bfloat16)
        cbv = jnp.zeros((1, NL), jnp.float32)
        for c in range(C_OUT):
            cbv = cbv + cb_in_ref[b, c] * ch_ref[c:c + 1, :]
        cb_ref[b] = cbv
    w1ab_ref[...] = jnp.zeros((NL, FC1N), jnp.bfloat16)
    for c in range(C_OUT):
        for i in range(H_POOL):
            dst = c * FRAME + i * W_CONV
            src = c * H_POOL * W_POOL + i * W_POOL
            blk_a = fc1T_ref[src:src + W_POOL, :].astype(jnp.bfloat16)
            blk_b = fc1T_ref[FEAT_HALF + src:FEAT_HALF + src + W_POOL, :]
            w1ab_ref[dst:dst + W_POOL, 0:HID1] = blk_a
            w1ab_ref[dst:dst + W_POOL, 128:128 + HID1] = blk_b.astype(jnp.bfloat16)


def _prep(conv_w, conv_b, fc1T):
    io, jo, ch, hi, wi = _index_rows()
    return pl.pallas_call(
        _prep_kernel,
        out_shape=(
            jax.ShapeDtypeStruct((2, PIX, NL), jnp.bfloat16),
            jax.ShapeDtypeStruct((2, 1, NL), jnp.float32),
            jax.ShapeDtypeStruct((NL, FC1N), jnp.bfloat16),
        ),
        in_specs=[
            pl.BlockSpec(memory_space=pltpu.MemorySpace.SMEM),
            pl.BlockSpec(memory_space=pltpu.MemorySpace.SMEM),
            pl.BlockSpec((FEAT_HALF * 2, HID1), lambda: (0, 0)),
            pl.BlockSpec((1, NL), lambda: (0, 0)),
            pl.BlockSpec((1, NL), lambda: (0, 0)),
            pl.BlockSpec((C_OUT, NL), lambda: (0, 0)),
            pl.BlockSpec((PIX, 1), lambda: (0, 0)),
            pl.BlockSpec((PIX, 1), lambda: (0, 0)),
        ],
        out_specs=(
            pl.BlockSpec((2, PIX, NL), lambda: (0, 0, 0)),
            pl.BlockSpec((2, 1, NL), lambda: (0, 0, 0)),
            pl.BlockSpec((NL, FC1N), lambda: (0, 0)),
        ),
    )(conv_w, conv_b, fc1T, io, jo, ch, hi, wi)


def _branch_kernel(x_ref, wb_ref, cb_ref, w1ab_ref, b1_ref,
                   w2_ref, b2_ref, w3_ref, b3_ref, o_ref):
    x = x_ref[...]
    z = jnp.dot(x, wb_ref[...],
                preferred_element_type=jnp.float32)
    z = jnp.maximum(z + cb_ref[...], 0.0)
    zb = z.astype(jnp.bfloat16)
    m = jnp.maximum(zb, jnp.concatenate(
        [zb[:, 1:], zb[:, :1]], axis=1))
    m = jnp.maximum(m, jnp.concatenate(
        [m[:, W_CONV:], m[:, :W_CONV]], axis=1))
    hsum = jnp.dot(m, w1ab_ref[...],
                   preferred_element_type=jnp.float32)
    ha = hsum[:, 0:HID1]
    hb = hsum[:, 128:128 + HID1]
    hbu = jnp.concatenate([hb[1:], hb[:1]], axis=0)
    h = jnp.maximum(ha + hbu + b1_ref[...], 0.0)
    h = jnp.dot(h.astype(jnp.bfloat16), w2_ref[...],
                preferred_element_type=jnp.float32)
    h = jnp.maximum(h + b2_ref[...], 0.0)
    o = jnp.dot(h.astype(jnp.bfloat16), w3_ref[...],
                preferred_element_type=jnp.float32)
    o_ref[...] = o + b3_ref[...]


def _run_branch(b, x, wb, cb, w1ab, b1, w2, b2, w3, b3):
    n = x.shape[0]
    nt = n // BS
    half = max(nt // 2, 1)
    grid = (nt // half, half)
    flops = 2 * n * (PIX * NL // 2 + NL * FC1N // 2 + HID1 * HID2 + HID2 * N_CLS)
    bytes_accessed = 2 * n * PIX + 4 * n * N_CLS
    return pl.pallas_call(
        _branch_kernel,
        out_shape=jax.ShapeDtypeStruct((n, N_CLS), jnp.float32),
        grid=grid,
        in_specs=[
            pl.BlockSpec((BS, PIX), lambda s, t, h=half: (s * h + t, 0)),
            pl.BlockSpec((None, PIX, NL), lambda s, t, b=b: (b, 0, 0)),
            pl.BlockSpec((None, 1, NL), lambda s, t, b=b: (b, 0, 0)),
            pl.BlockSpec((NL, FC1N), lambda s, t: (0, 0)),
            pl.BlockSpec((1, HID1), lambda s, t: (0, 0)),
            pl.BlockSpec((HID1, HID2), lambda s, t: (0, 0)),
            pl.BlockSpec((1, HID2), lambda s, t: (0, 0)),
            pl.BlockSpec((HID2, N_CLS), lambda s, t: (0, 0)),
            pl.BlockSpec((1, N_CLS), lambda s, t: (0, 0)),
        ],
        out_specs=pl.BlockSpec((BS, N_CLS), lambda s, t, h=half: (s * h + t, 0)),
        compiler_params=pltpu.CompilerParams(
            dimension_semantics=("parallel", "parallel")),
        cost_estimate=pl.CostEstimate(flops=flops, transcendentals=0,
                                      bytes_accessed=bytes_accessed),
    )(x, wb, cb, w1ab, b1, w2, b2, w3, b3)


@functools.partial(jax.jit, static_argnames=())
def kernel(x1, x2, conv_w, conv_b, fc1_w, fc1_b, fc2_w, fc2_b, fc3_w, fc3_b):
    n = x1.shape[0]
    n_pad = (n + BS - 1) // BS * BS

    wb, cb, w1ab = _prep(conv_w, conv_b, fc1_w.T)
    b1 = fc1_b.reshape(1, HID1)
    w2 = fc2_w.T.astype(jnp.bfloat16)
    b2 = fc2_b.reshape(1, HID2)
    w3 = fc3_w.T.astype(jnp.bfloat16)
    b3 = fc3_b.reshape(1, N_CLS)

    outs = []
    for b in range(2):
        x = (x1 if b == 0 else x2).astype(jnp.bfloat16).reshape(n, PIX)
        if n_pad != n:
            x = jnp.pad(x, ((0, n_pad - n), (0, 0)))
        o = _run_branch(b, x, wb, cb, w1ab, b1, w2, b2, w3, b3)
        outs.append(o[:n].reshape(n // 2, 2 * N_CLS)[:, :N_CLS])
    return jnp.concatenate(outs, axis=0)


# speedup vs baseline: 21.4406x; 1.0079x over previous
import functools

import jax
import jax.numpy as jnp
import numpy as np
from jax.experimental import pallas as pl
from jax.experimental.pallas import tpu as pltpu

C_OUT = 5
KW = 3
H_IN, W_IN = 20, 20
PIX = H_IN * W_IN
H_CONV, W_CONV = H_IN - KW + 1, W_IN - KW + 1
H_POOL, W_POOL = H_CONV - 1, W_CONV - 1
FRAME = H_CONV * W_CONV
NL = 1664
FEAT_HALF = C_OUT * H_POOL * W_POOL
HID1, HID2, N_CLS = 64, 16, 4
FC1N = 256
BS = 2048


def _index_rows():
    io = np.full((1, NL), -1000.0, np.float32)
    jo = np.full((1, NL), -1000.0, np.float32)
    ch = np.zeros((C_OUT, NL), np.float32)
    for c in range(C_OUT):
        ch[c, c * FRAME:(c + 1) * FRAME] = 1.0
        for i in range(H_CONV):
            for j in range(W_CONV):
                q = c * FRAME + i * W_CONV + j
                io[0, q] = i
                jo[0, q] = j
    hi = np.repeat(np.arange(H_IN), W_IN).astype(np.float32).reshape(PIX, 1)
    wi = np.tile(np.arange(W_IN), H_IN).astype(np.float32).reshape(PIX, 1)
    return (jnp.asarray(io), jnp.asarray(jo), jnp.asarray(ch),
            jnp.asarray(hi), jnp.asarray(wi))


def _prep_kernel(cw_ref, cb_in_ref, fc1T_ref, io_ref, jo_ref, ch_ref,
                 hi_ref, wi_ref, wb_ref, cb_ref, w1ab_ref):
    dh = hi_ref[...] - io_ref[...]
    dw = wi_ref[...] - jo_ref[...]
    for b in range(2):
        acc = jnp.zeros((PIX, NL), jnp.float32)
        for d in range(KW):
            for e in range(KW):
                tv = jnp.zeros((1, NL), jnp.float32)
                for c in range(C_OUT):
                    tv = tv + cw_ref[b, c * 9 + d * 3 + e] * ch_ref[c:c + 1, :]
                band = jnp.logical_and(dh == float(d), dw == float(e))
                acc = acc + jnp.where(band, tv, 0.0)
        wb_ref[b] = acc.astype(jnp.bfloat16)
        cbv = jnp.zeros((1, NL), jnp.float32)
        for c in range(C_OUT):
            cbv = cbv + cb_in_ref[b, c] * ch_ref[c:c + 1, :]
        cb_ref[b] = cbv
    w1ab_ref[...] = jnp.zeros((NL, FC1N), jnp.bfloat16)
    for c in range(C_OUT):
        for i in range(H_POOL):
            dst = c * FRAME + i * W_CONV
            src = c * H_POOL * W_POOL + i * W_POOL
            blk_a = fc1T_ref[src:src + W_POOL, :].astype(jnp.bfloat16)
            blk_b = fc1T_ref[FEAT_HALF + src:FEAT_HALF + src + W_POOL, :]
            w1ab_ref[dst:dst + W_POOL, 0:HID1] = blk_a
            w1ab_ref[dst:dst + W_POOL, 128:128 + HID1] = blk_b.astype(jnp.bfloat16)


def _prep(conv_w, conv_b, fc1T):
    io, jo, ch, hi, wi = _index_rows()
    return pl.pallas_call(
        _prep_kernel,
        out_shape=(
            jax.ShapeDtypeStruct((2, PIX, NL), jnp.bfloat16),
            jax.ShapeDtypeStruct((2, 1, NL), jnp.float32),
            jax.ShapeDtypeStruct((NL, FC1N), jnp.bfloat16),
        ),
        in_specs=[
            pl.BlockSpec(memory_space=pltpu.MemorySpace.SMEM),
            pl.BlockSpec(memory_space=pltpu.MemorySpace.SMEM),
            pl.BlockSpec((FEAT_HALF * 2, HID1), lambda: (0, 0)),
            pl.BlockSpec((1, NL), lambda: (0, 0)),
            pl.BlockSpec((1, NL), lambda: (0, 0)),
            pl.BlockSpec((C_OUT, NL), lambda: (0, 0)),
            pl.BlockSpec((PIX, 1), lambda: (0, 0)),
            pl.BlockSpec((PIX, 1), lambda: (0, 0)),
        ],
        out_specs=(
            pl.BlockSpec((2, PIX, NL), lambda: (0, 0, 0)),
            pl.BlockSpec((2, 1, NL), lambda: (0, 0, 0)),
            pl.BlockSpec((NL, FC1N), lambda: (0, 0)),
        ),
    )(conv_w, conv_b, fc1T, io, jo, ch, hi, wi)


def _branch_kernel(x_ref, wb_ref, cb_ref, w1ab_ref, b1_ref,
                   w2_ref, b2_ref, w3_ref, b3_ref, o_ref):
    x = x_ref[...]
    z = jnp.dot(x, wb_ref[...],
                preferred_element_type=jnp.float32)
    z = jnp.maximum(z + cb_ref[...], 0.0)
    zb = z.astype(jnp.bfloat16)
    m = jnp.maximum(zb, jnp.concatenate(
        [zb[:, 1:], zb[:, :1]], axis=1))
    m = jnp.maximum(m, jnp.concatenate(
        [m[:, W_CONV:], m[:, :W_CONV]], axis=1))
    hsum = jnp.dot(m, w1ab_ref[...],
                   preferred_element_type=jnp.float32)
    ha = hsum[:, 0:HID1]
    hb = hsum[:, 128:128 + HID1]
    hbu = jnp.concatenate([hb[1:], hb[:1]], axis=0)
    h = jnp.maximum(ha + hbu + b1_ref[...], 0.0)
    h = jnp.dot(h.astype(jnp.bfloat16), w2_ref[...],
                preferred_element_type=jnp.float32)
    h = jnp.maximum(h + b2_ref[...], 0.0)
    o = jnp.dot(h.astype(jnp.bfloat16), w3_ref[...],
                preferred_element_type=jnp.float32)
    o_ref[...] = o + b3_ref[...]


def _run_branch(b, x, wb, cb, w1ab, b1, w2, b2, w3, b3):
    n = x.shape[0]
    nt = n // BS
    half = max(nt // 2, 1)
    grid = (nt // half, half)
    flops = 2 * n * (PIX * NL // 2 + NL * FC1N // 2 + HID1 * HID2 + HID2 * N_CLS)
    bytes_accessed = 2 * n * PIX + 4 * n * N_CLS
    return pl.pallas_call(
        _branch_kernel,
        out_shape=jax.ShapeDtypeStruct((n, N_CLS), jnp.float32),
        grid=grid,
        in_specs=[
            pl.BlockSpec((BS, PIX), lambda s, t, h=half: (s * h + t, 0)),
            pl.BlockSpec((None, PIX, NL), lambda s, t, b=b: (b, 0, 0)),
            pl.BlockSpec((None, 1, NL), lambda s, t, b=b: (b, 0, 0)),
            pl.BlockSpec((NL, FC1N), lambda s, t: (0, 0)),
            pl.BlockSpec((1, HID1), lambda s, t: (0, 0)),
            pl.BlockSpec((HID1, HID2), lambda s, t: (0, 0)),
            pl.BlockSpec((1, HID2), lambda s, t: (0, 0)),
            pl.BlockSpec((HID2, N_CLS), lambda s, t: (0, 0)),
            pl.BlockSpec((1, N_CLS), lambda s, t: (0, 0)),
        ],
        out_specs=pl.BlockSpec((BS, N_CLS), lambda s, t, h=half: (s * h + t, 0)),
        compiler_params=pltpu.CompilerParams(
            dimension_semantics=("parallel", "parallel")),
        cost_estimate=pl.CostEstimate(flops=flops, transcendentals=0,
                                      bytes_accessed=bytes_accessed),
    )(x, wb, cb, w1ab, b1, w2, b2, w3, b3)


@functools.partial(jax.jit, static_argnames=())
def kernel(x1, x2, conv_w, conv_b, fc1_w, fc1_b, fc2_w, fc2_b, fc3_w, fc3_b):
    n = x1.shape[0]
    n_pad = (n + BS - 1) // BS * BS

    wb, cb, w1ab = _prep(conv_w, conv_b, fc1_w.T)
    b1 = fc1_b.reshape(1, HID1)
    w2 = fc2_w.T.astype(jnp.bfloat16)
    b2 = fc2_b.reshape(1, HID2)
    w3 = fc3_w.T.astype(jnp.bfloat16)
    b3 = fc3_b.reshape(1, N_CLS)

    outs = []
    for b in range(2):
        x = (x1 if b == 0 else x2).astype(jnp.bfloat16).reshape(n, PIX)
        if n_pad != n:
            x = jnp.pad(x, ((0, n_pad - n), (0, 0)))
        o = _run_branch(b, x, wb, cb, w1ab, b1, w2, b2, w3, b3)
        outs.append(o[:n].reshape(n // 2, 2 * N_CLS)[:, :N_CLS])
    return jnp.concatenate(outs, axis=0)
